# Initial kernel scaffold; baseline (speedup 1.0000x reference)
import functools

import jax
import jax.numpy as jnp
from jax import lax
from jax.experimental import pallas as pl
from jax.experimental.pallas import tpu as pltpu
from jax.experimental.pallas import tpu_sc as plsc

_NC = 2
_NS = 16
_NW = _NC * _NS


def _pick_chunk(epw):
    for c in range(128, 7, -8):
        if epw % c == 0:
            return c
    raise ValueError(f"no valid chunk size for {epw} edges per worker")


def _make_sc_edge(n, e, wt):
    wa = wt // 2
    epw = e // _NW
    ch = _pick_chunk(epw)
    nch = epw // ch
    rpt = n // _NS
    zrows = 125
    nq = wa // 16

    mesh = plsc.VectorSubcoreMesh(core_axis_name="c", subcore_axis_name="s")

    @functools.partial(
        pl.kernel,
        out_type=jax.ShapeDtypeStruct((_NC, n, wt), jnp.float32),
        mesh=mesh,
        scratch_types=[
            pltpu.VMEM((nch, ch), jnp.int32),
            pltpu.VMEM((nch, ch), jnp.int32),
            pltpu.VMEM((ch, wt), jnp.float32),
            pltpu.VMEM((ch, wa), jnp.float32),
            pltpu.VMEM((125, wt), jnp.float32),
            pltpu.VMEM_SHARED((n, wt), jnp.float32),
            pltpu.SemaphoreType.DMA,
            pltpu.SemaphoreType.DMA,
        ],
    )
    def sc_edge(table_hbm, ad_hbm, src_hbm, dst_hbm, out_hbm,
                src_v, dst_v, rows_v, ad_v, zb_v, acc, sem1, sem2):
        cid = lax.axis_index("c")
        sid = lax.axis_index("s")
        wid = cid * _NS + sid

        pltpu.sync_copy(src_hbm.at[wid], src_v)
        pltpu.sync_copy(dst_hbm.at[wid], dst_v)

        zero16 = jnp.zeros((16,), jnp.float32)
        nqt = wt // 16

        @pl.loop(0, zrows * nqt)
        def _zfill(i):
            zb_v[i // nqt, pl.ds((i % nqt) * 16, 16)] = zero16

        nzc = rpt // zrows
        rem = rpt - nzc * zrows
        for z in range(nzc):
            pltpu.sync_copy(zb_v, acc.at[pl.ds(sid * rpt + z * zrows, zrows)])
        if rem:
            pltpu.sync_copy(zb_v.at[pl.ds(0, rem)],
                            acc.at[pl.ds(sid * rpt + nzc * zrows, rem)])
        plsc.subcore_barrier()

        @pl.loop(0, nch)
        def _chunk(j):
            g1 = pltpu.async_copy(table_hbm.at[src_v.at[j]], rows_v, sem1)
            g2 = pltpu.async_copy(ad_hbm.at[dst_v.at[j]], ad_v, sem2)
            g1.wait()
            g2.wait()

            @pl.loop(0, ch)
            def _edge(t):
                for q in range(nq):
                    asrc = rows_v[t, pl.ds(wa + q * 16, 16)]
                    adst = ad_v[t, pl.ds(q * 16, 16)]
                    al = asrc + adst
                    al = jnp.where(al > 0.0, al, al * 0.2)
                    w = jnp.exp(al)
                    xw = rows_v[t, pl.ds(q * 16, 16)]
                    rows_v[t, pl.ds(q * 16, 16)] = xw * w
                    rows_v[t, pl.ds(wa + q * 16, 16)] = w

            pltpu.sync_copy(rows_v, acc.at[dst_v.at[j]], add=True)

        plsc.subcore_barrier()
        pltpu.sync_copy(acc.at[pl.ds(sid * rpt, rpt)],
                        out_hbm.at[cid, pl.ds(sid * rpt, rpt)])

    return sc_edge


def _leaky(x):
    return jnp.where(x >= 0.0, x, x * 0.2)


def _tc_pre_body(x_ref, w1_ref, ae_ref, be_ref, table_ref, ad_ref, self_ref):
    xw = jnp.dot(x_ref[...], w1_ref[...], preferred_element_type=jnp.float32)
    asrc = jnp.dot(xw, ae_ref[...], preferred_element_type=jnp.float32)
    adst = jnp.dot(xw, be_ref[...], preferred_element_type=jnp.float32)
    table_ref[...] = jnp.concatenate([xw, asrc], axis=1)
    ad_ref[...] = adst
    w = jnp.exp(_leaky(asrc + adst))
    self_ref[...] = jnp.concatenate([xw * w, w], axis=1)


def _tc_mid_body(acc_ref, self_ref, b1_ref, w2p_ref, sv32_ref, dv16_ref,
                 table_ref, ad_ref, self2_ref):
    a = acc_ref[0] + acc_ref[1] + self_ref[...]
    h = a[:, :64] / (a[:, 64:] + 1e-16) + b1_ref[...]
    h = jnp.where(h > 0.0, h, jnp.expm1(jnp.minimum(h, 0.0)))
    xw2 = jnp.dot(h, w2p_ref[...], preferred_element_type=jnp.float32)
    t2 = xw2 + jnp.dot(xw2, sv32_ref[...], preferred_element_type=jnp.float32)
    adx = jnp.dot(xw2, dv16_ref[...], preferred_element_type=jnp.float32)
    table_ref[...] = t2
    ad_ref[...] = adx
    w = jnp.exp(_leaky(t2[:, 16:] + adx))
    self2_ref[...] = jnp.concatenate([xw2[:, :16] * w, w], axis=1)


def _tc_post_body(acc_ref, self_ref, b2_ref, o_ref):
    a = acc_ref[0] + acc_ref[1] + self_ref[...]
    o = a[:, 0:2] / (a[:, 16:18] + 1e-16) + b2_ref[...]
    m = jnp.max(o, axis=1, keepdims=True)
    o_ref[...] = o - m - jnp.log(jnp.sum(jnp.exp(o - m), axis=1, keepdims=True))


def _expand_att(a):
    hh, cc = a.shape
    eye = jnp.eye(hh, dtype=a.dtype)
    t = eye[:, None, :] * a.T[None, :, :]
    t4 = jnp.broadcast_to(t[:, :, :, None], (hh, cc, hh, cc))
    return t4.reshape(hh * cc, hh * cc)


def kernel(x, edge_index, W1, att_src1, att_dst1, bias1,
           W2, att_src2, att_dst2, bias2):
    n, f_in = x.shape
    e = edge_index.shape[1]
    h1, c1 = att_src1.shape[1], att_src1.shape[2]
    c2 = att_src2.shape[2]
    d1 = h1 * c1

    ae1 = _expand_att(att_src1.reshape(h1, c1))
    be1 = _expand_att(att_dst1.reshape(h1, c1))
    w2p = jnp.zeros((d1, 32), jnp.float32).at[:, :c2].set(W2)
    sv = jnp.zeros((32,), jnp.float32).at[:c2].set(att_src2.reshape(c2))
    dv = jnp.zeros((32,), jnp.float32).at[:c2].set(att_dst2.reshape(c2))
    sv32 = jnp.concatenate(
        [jnp.zeros((32, 16), jnp.float32),
         jnp.broadcast_to(sv[:, None], (32, 16))], axis=1)
    dv16 = jnp.broadcast_to(dv[:, None], (32, 16))
    b1r = bias1.reshape(1, d1)
    b2r = bias2.reshape(1, c2)

    epw = e // _NW
    ch = _pick_chunk(epw)
    nch = epw // ch
    src3 = edge_index[0].reshape(_NW, nch, ch)
    dst3 = edge_index[1].reshape(_NW, nch, ch)

    r = 1000
    grid = (n // r,)
    table1, ad1, self1 = pl.pallas_call(
        _tc_pre_body,
        grid=grid,
        in_specs=[
            pl.BlockSpec((r, f_in), lambda i: (i, 0)),
            pl.BlockSpec((f_in, d1), lambda i: (0, 0)),
            pl.BlockSpec((d1, d1), lambda i: (0, 0)),
            pl.BlockSpec((d1, d1), lambda i: (0, 0)),
        ],
        out_specs=[
            pl.BlockSpec((r, 2 * d1), lambda i: (i, 0)),
            pl.BlockSpec((r, d1), lambda i: (i, 0)),
            pl.BlockSpec((r, 2 * d1), lambda i: (i, 0)),
        ],
        out_shape=[
            jax.ShapeDtypeStruct((n, 2 * d1), jnp.float32),
            jax.ShapeDtypeStruct((n, d1), jnp.float32),
            jax.ShapeDtypeStruct((n, 2 * d1), jnp.float32),
        ],
    )(x, W1, ae1, be1)

    acc1 = _make_sc_edge(n, e, 2 * d1)(table1, ad1, src3, dst3)

    table2, ad2, self2 = pl.pallas_call(
        _tc_mid_body,
        grid=grid,
        in_specs=[
            pl.BlockSpec((_NC, r, 2 * d1), lambda i: (0, i, 0)),
            pl.BlockSpec((r, 2 * d1), lambda i: (i, 0)),
            pl.BlockSpec((1, d1), lambda i: (0, 0)),
            pl.BlockSpec((d1, 32), lambda i: (0, 0)),
            pl.BlockSpec((32, 32), lambda i: (0, 0)),
            pl.BlockSpec((32, 16), lambda i: (0, 0)),
        ],
        out_specs=[
            pl.BlockSpec((r, 32), lambda i: (i, 0)),
            pl.BlockSpec((r, 16), lambda i: (i, 0)),
            pl.BlockSpec((r, 32), lambda i: (i, 0)),
        ],
        out_shape=[
            jax.ShapeDtypeStruct((n, 32), jnp.float32),
            jax.ShapeDtypeStruct((n, 16), jnp.float32),
            jax.ShapeDtypeStruct((n, 32), jnp.float32),
        ],
    )(acc1, self1, b1r, w2p, sv32, dv16)

    acc2 = _make_sc_edge(n, e, 32)(table2, ad2, src3, dst3)

    out = pl.pallas_call(
        _tc_post_body,
        grid=grid,
        in_specs=[
            pl.BlockSpec((_NC, r, 32), lambda i: (0, i, 0)),
            pl.BlockSpec((r, 32), lambda i: (i, 0)),
            pl.BlockSpec((1, c2), lambda i: (0, 0)),
        ],
        out_specs=pl.BlockSpec((r, c2), lambda i: (i, 0)),
        out_shape=jax.ShapeDtypeStruct((n, c2), jnp.float32),
    )(acc2, self2, b2r)

    return out


# speedup vs baseline: 47.1708x; 47.1708x over previous
import functools

import jax
import jax.numpy as jnp
from jax import lax
from jax.experimental import pallas as pl
from jax.experimental.pallas import tpu as pltpu
from jax.experimental.pallas import tpu_sc as plsc

_NC = 2
_NS = 16
_NW = _NC * _NS


def _pick_chunk(epw):
    for c in range(128, 7, -8):
        if epw % c == 0:
            return c
    raise ValueError(f"no valid chunk size for {epw} edges per worker")


def _make_sc_edge(n, e, wt):
    wa = wt // 2
    epw = e // _NW
    ch = _pick_chunk(epw)
    nch = epw // ch
    rpt = (n // _NS) // 8 * 8
    tail = n - rpt * _NS
    zrows = 104
    nq = wa // 16

    mesh = plsc.VectorSubcoreMesh(core_axis_name="c", subcore_axis_name="s")

    @functools.partial(
        pl.kernel,
        out_type=jax.ShapeDtypeStruct((_NC, n, wt), jnp.float32),
        mesh=mesh,
        scratch_types=[
            pltpu.VMEM((nch, ch), jnp.int32),
            pltpu.VMEM((nch, ch), jnp.int32),
            pltpu.VMEM((ch, wt), jnp.float32),
            pltpu.VMEM((ch, wa), jnp.float32),
            pltpu.VMEM((zrows, wt), jnp.float32),
            pltpu.VMEM_SHARED((n, wt), jnp.float32),
            pltpu.SemaphoreType.DMA,
            pltpu.SemaphoreType.DMA,
        ],
        compiler_params=pltpu.CompilerParams(use_tc_tiling_on_sc=False),
    )
    def sc_edge(table_hbm, ad_hbm, src_hbm, dst_hbm, out_hbm,
                src_v, dst_v, rows_v, ad_v, zb_v, acc, sem1, sem2):
        cid = lax.axis_index("c")
        sid = lax.axis_index("s")
        wid = cid * _NS + sid

        pltpu.sync_copy(src_hbm.at[wid], src_v)
        pltpu.sync_copy(dst_hbm.at[wid], dst_v)

        zero16 = jnp.zeros((16,), jnp.float32)
        nqt = wt // 16

        @pl.loop(0, zrows * nqt)
        def _zfill(i):
            zb_v[i // nqt, pl.ds((i % nqt) * 16, 16)] = zero16

        nzc = rpt // zrows
        rem = rpt - nzc * zrows
        for z in range(nzc):
            pltpu.sync_copy(zb_v, acc.at[pl.ds(sid * rpt + z * zrows, zrows)])
        if rem:
            pltpu.sync_copy(zb_v.at[pl.ds(0, rem)],
                            acc.at[pl.ds(sid * rpt + nzc * zrows, rem)])
        if tail:
            @pl.when(sid == _NS - 1)
            def _ztail():
                pltpu.sync_copy(zb_v.at[pl.ds(0, tail)],
                                acc.at[pl.ds(_NS * rpt, tail)])
        plsc.subcore_barrier()

        @pl.loop(0, nch)
        def _chunk(j):
            g1 = pltpu.async_copy(table_hbm.at[src_v.at[j]], rows_v, sem1)
            g2 = pltpu.async_copy(ad_hbm.at[dst_v.at[j]], ad_v, sem2)
            g1.wait()
            g2.wait()

            @pl.loop(0, ch)
            def _edge(t):
                for q in range(nq):
                    asrc = rows_v[t, pl.ds(wa + q * 16, 16)]
                    adst = ad_v[t, pl.ds(q * 16, 16)]
                    al = asrc + adst
                    al = jnp.where(al > 0.0, al, al * 0.2)
                    w = jnp.exp(al)
                    xw = rows_v[t, pl.ds(q * 16, 16)]
                    rows_v[t, pl.ds(q * 16, 16)] = xw * w
                    rows_v[t, pl.ds(wa + q * 16, 16)] = w

            pltpu.sync_copy(rows_v, acc.at[dst_v.at[j]], add=True)

        plsc.subcore_barrier()
        pltpu.sync_copy(acc.at[pl.ds(sid * rpt, rpt)],
                        out_hbm.at[cid, pl.ds(sid * rpt, rpt)])
        if tail:
            @pl.when(sid == _NS - 1)
            def _otail():
                pltpu.sync_copy(acc.at[pl.ds(_NS * rpt, tail)],
                                out_hbm.at[cid, pl.ds(_NS * rpt, tail)])

    return sc_edge


def _leaky(x):
    return jnp.where(x >= 0.0, x, x * 0.2)


def _tc_pre_body(x_ref, w1_ref, ae_ref, be_ref, table_ref, ad_ref, self_ref):
    xw = jnp.dot(x_ref[...], w1_ref[...], preferred_element_type=jnp.float32)
    asrc = jnp.dot(xw, ae_ref[...], preferred_element_type=jnp.float32)
    adst = jnp.dot(xw, be_ref[...], preferred_element_type=jnp.float32)
    table_ref[...] = jnp.concatenate([xw, asrc], axis=1)
    ad_ref[...] = adst
    w = jnp.exp(_leaky(asrc + adst))
    self_ref[...] = jnp.concatenate([xw * w, w], axis=1)


def _tc_mid_body(acc_ref, self_ref, b1_ref, w2p_ref, sv32_ref, dv16_ref,
                 table_ref, ad_ref, self2_ref):
    a = acc_ref[0] + acc_ref[1] + self_ref[...]
    h = a[:, :64] / (a[:, 64:] + 1e-16) + b1_ref[...]
    h = jnp.where(h > 0.0, h, jnp.exp(jnp.minimum(h, 0.0)) - 1.0)
    xw2 = jnp.dot(h, w2p_ref[...], preferred_element_type=jnp.float32)
    t2 = xw2 + jnp.dot(xw2, sv32_ref[...], preferred_element_type=jnp.float32)
    adx = jnp.dot(xw2, dv16_ref[...], preferred_element_type=jnp.float32)
    table_ref[...] = t2
    ad_ref[...] = adx
    w = jnp.exp(_leaky(t2[:, 16:] + adx))
    self2_ref[...] = jnp.concatenate([xw2[:, :16] * w, w], axis=1)


def _tc_post_body(acc_ref, self_ref, b2_ref, o_ref):
    a = acc_ref[0] + acc_ref[1] + self_ref[...]
    o = a[:, 0:2] / (a[:, 16:18] + 1e-16) + b2_ref[...]
    m = jnp.max(o, axis=1, keepdims=True)
    o_ref[...] = o - m - jnp.log(jnp.sum(jnp.exp(o - m), axis=1, keepdims=True))


def _expand_att(a):
    hh, cc = a.shape
    eye = jnp.eye(hh, dtype=a.dtype)
    t = eye[:, None, :] * a.T[None, :, :]
    t4 = jnp.broadcast_to(t[:, :, :, None], (hh, cc, hh, cc))
    return t4.reshape(hh * cc, hh * cc)


def kernel(x, edge_index, W1, att_src1, att_dst1, bias1,
           W2, att_src2, att_dst2, bias2):
    n, f_in = x.shape
    e = edge_index.shape[1]
    h1, c1 = att_src1.shape[1], att_src1.shape[2]
    c2 = att_src2.shape[2]
    d1 = h1 * c1

    ae1 = _expand_att(att_src1.reshape(h1, c1))
    be1 = _expand_att(att_dst1.reshape(h1, c1))
    w2p = jnp.zeros((d1, 32), jnp.float32).at[:, :c2].set(W2)
    sv = jnp.zeros((32,), jnp.float32).at[:c2].set(att_src2.reshape(c2))
    dv = jnp.zeros((32,), jnp.float32).at[:c2].set(att_dst2.reshape(c2))
    sv32 = jnp.concatenate(
        [jnp.zeros((32, 16), jnp.float32),
         jnp.broadcast_to(sv[:, None], (32, 16))], axis=1)
    dv16 = jnp.broadcast_to(dv[:, None], (32, 16))
    b1r = bias1.reshape(1, d1)
    b2r = bias2.reshape(1, c2)

    epw = e // _NW
    ch = _pick_chunk(epw)
    nch = epw // ch
    src3 = edge_index[0].reshape(_NW, nch, ch)
    dst3 = edge_index[1].reshape(_NW, nch, ch)

    r = 1000
    grid = (n // r,)
    table1, ad1, self1 = pl.pallas_call(
        _tc_pre_body,
        grid=grid,
        in_specs=[
            pl.BlockSpec((r, f_in), lambda i: (i, 0)),
            pl.BlockSpec((f_in, d1), lambda i: (0, 0)),
            pl.BlockSpec((d1, d1), lambda i: (0, 0)),
            pl.BlockSpec((d1, d1), lambda i: (0, 0)),
        ],
        out_specs=[
            pl.BlockSpec((r, 2 * d1), lambda i: (i, 0)),
            pl.BlockSpec((r, d1), lambda i: (i, 0)),
            pl.BlockSpec((r, 2 * d1), lambda i: (i, 0)),
        ],
        out_shape=[
            jax.ShapeDtypeStruct((n, 2 * d1), jnp.float32),
            jax.ShapeDtypeStruct((n, d1), jnp.float32),
            jax.ShapeDtypeStruct((n, 2 * d1), jnp.float32),
        ],
    )(x, W1, ae1, be1)

    acc1 = _make_sc_edge(n, e, 2 * d1)(table1, ad1, src3, dst3)

    table2, ad2, self2 = pl.pallas_call(
        _tc_mid_body,
        grid=grid,
        in_specs=[
            pl.BlockSpec((_NC, r, 2 * d1), lambda i: (0, i, 0)),
            pl.BlockSpec((r, 2 * d1), lambda i: (i, 0)),
            pl.BlockSpec((1, d1), lambda i: (0, 0)),
            pl.BlockSpec((d1, 32), lambda i: (0, 0)),
            pl.BlockSpec((32, 32), lambda i: (0, 0)),
            pl.BlockSpec((32, 16), lambda i: (0, 0)),
        ],
        out_specs=[
            pl.BlockSpec((r, 32), lambda i: (i, 0)),
            pl.BlockSpec((r, 16), lambda i: (i, 0)),
            pl.BlockSpec((r, 32), lambda i: (i, 0)),
        ],
        out_shape=[
            jax.ShapeDtypeStruct((n, 32), jnp.float32),
            jax.ShapeDtypeStruct((n, 16), jnp.float32),
            jax.ShapeDtypeStruct((n, 32), jnp.float32),
        ],
    )(acc1, self1, b1r, w2p, sv32, dv16)

    acc2 = _make_sc_edge(n, e, 32)(table2, ad2, src3, dst3)

    out = pl.pallas_call(
        _tc_post_body,
        grid=grid,
        in_specs=[
            pl.BlockSpec((_NC, r, 32), lambda i: (0, i, 0)),
            pl.BlockSpec((r, 32), lambda i: (i, 0)),
            pl.BlockSpec((1, c2), lambda i: (0, 0)),
        ],
        out_specs=pl.BlockSpec((r, c2), lambda i: (i, 0)),
        out_shape=jax.ShapeDtypeStruct((n, c2), jnp.float32),
    )(acc2, self2, b2r)

    return out


# speedup vs baseline: 79.9422x; 1.6947x over previous
import functools

import jax
import jax.numpy as jnp
from jax import lax
from jax.experimental import pallas as pl
from jax.experimental.pallas import tpu as pltpu
from jax.experimental.pallas import tpu_sc as plsc

_NC = 2
_NS = 16
_NW = _NC * _NS


def _pick_chunk(epw):
    for c in range(128, 7, -8):
        if epw % c == 0:
            return c
    raise ValueError(f"no valid chunk size for {epw} edges per worker")


def _make_sc_edge(n, e, wx):
    wt = wx + 16
    epw = e // _NW
    ch = _pick_chunk(epw)
    nch = epw // ch
    rpt = (n // _NS) // 8 * 8
    tail = n - rpt * _NS
    zrows = 104
    nq = wx // 16

    mesh = plsc.VectorSubcoreMesh(core_axis_name="c", subcore_axis_name="s")

    @functools.partial(
        pl.kernel,
        out_type=jax.ShapeDtypeStruct((_NC, n, wt), jnp.float32),
        mesh=mesh,
        scratch_types=[
            pltpu.VMEM((nch, ch), jnp.int32),
            pltpu.VMEM((nch, ch), jnp.int32),
            pltpu.VMEM((ch, wt), jnp.float32),
            pltpu.VMEM((ch, wt), jnp.float32),
            pltpu.VMEM((ch, 16), jnp.float32),
            pltpu.VMEM((ch, 16), jnp.float32),
            pltpu.VMEM((zrows, wt), jnp.float32),
            pltpu.VMEM_SHARED((n, wt), jnp.float32),
            pltpu.SemaphoreType.DMA,
            pltpu.SemaphoreType.DMA,
            pltpu.SemaphoreType.DMA,
            pltpu.SemaphoreType.DMA,
        ],
        compiler_params=pltpu.CompilerParams(use_tc_tiling_on_sc=False,
                                             needs_layout_passes=False),
    )
    def sc_edge(table_hbm, ad_hbm, src_hbm, dst_hbm, out_hbm,
                src_v, dst_v, rows0, rows1, ad0, ad1, zb_v, acc,
                sgt0, sga0, sgt1, sga1):
        cid = lax.axis_index("c")
        sid = lax.axis_index("s")
        wid = cid * _NS + sid
        bufs = ((rows0, ad0, sgt0, sga0), (rows1, ad1, sgt1, sga1))

        pltpu.sync_copy(src_hbm.at[wid], src_v)
        pltpu.sync_copy(dst_hbm.at[wid], dst_v)

        zero16 = jnp.zeros((16,), jnp.float32)
        nqt = wt // 16

        @pl.loop(0, zrows * nqt)
        def _zfill(i):
            zb_v[i // nqt, pl.ds((i % nqt) * 16, 16)] = zero16

        nzc = rpt // zrows
        rem = rpt - nzc * zrows
        for z in range(nzc):
            pltpu.sync_copy(zb_v, acc.at[pl.ds(sid * rpt + z * zrows, zrows)])
        if rem:
            pltpu.sync_copy(zb_v.at[pl.ds(0, rem)],
                            acc.at[pl.ds(sid * rpt + nzc * zrows, rem)])
        if tail:
            @pl.when(sid == _NS - 1)
            def _ztail():
                pltpu.sync_copy(zb_v.at[pl.ds(0, tail)],
                                acc.at[pl.ds(_NS * rpt, tail)])
        plsc.subcore_barrier()

        pats = [lax.shift_right_logical(lax.iota(jnp.int32, 16) + 16 * q, 3)
                for q in range(nq)]

        def issue(c, b):
            rb, ab, st, sa = bufs[b]
            pltpu.async_copy(table_hbm.at[src_v.at[c]], rb, st)
            pltpu.async_copy(ad_hbm.at[dst_v.at[c]], ab, sa)

        def wait_g(c, b):
            rb, ab, st, sa = bufs[b]
            pltpu.make_async_copy(table_hbm.at[src_v.at[c]], rb, st).wait()
            pltpu.make_async_copy(ad_hbm.at[dst_v.at[c]], ab, sa).wait()

        def compute_scatter(c, b):
            rb, ab, _, _ = bufs[b]

            @pl.loop(0, ch, unroll=4)
            def _edge(t):
                al = rb[t, pl.ds(wx, 16)] + ab[t, :]
                al = jnp.where(al > 0.0, al, al * 0.2)
                w = jnp.exp(al)
                rb[t, pl.ds(wx, 16)] = w
                t16 = jnp.full((16,), t, jnp.int32)
                for q in range(nq):
                    wq = w if nq == 1 else plsc.load_gather(
                        rb, [t16, pats[q] + wx])
                    rb[t, pl.ds(q * 16, 16)] = rb[t, pl.ds(q * 16, 16)] * wq

            pltpu.sync_copy(rb, acc.at[dst_v.at[c]], add=True)

        issue(0, 0)

        @pl.loop(0, nch, step=2)
        def _pair(j):
            @pl.when(j + 1 < nch)
            def _():
                issue(j + 1, 1)
            wait_g(j, 0)
            compute_scatter(j, 0)

            @pl.when(j + 2 < nch)
            def _():
                issue(j + 2, 0)

            @pl.when(j + 1 < nch)
            def _():
                wait_g(j + 1, 1)
                compute_scatter(j + 1, 1)

        plsc.subcore_barrier()
        pltpu.sync_copy(acc.at[pl.ds(sid * rpt, rpt)],
                        out_hbm.at[cid, pl.ds(sid * rpt, rpt)])
        if tail:
            @pl.when(sid == _NS - 1)
            def _otail():
                pltpu.sync_copy(acc.at[pl.ds(_NS * rpt, tail)],
                                out_hbm.at[cid, pl.ds(_NS * rpt, tail)])

    return sc_edge


def _leaky(x):
    return jnp.where(x >= 0.0, x, x * 0.2)


def _tc_pre_body(x_ref, w1_ref, ae_ref, be_ref, xp_ref,
                 table_ref, ad_ref, self_ref):
    xw = jnp.dot(x_ref[...], w1_ref[...], preferred_element_type=jnp.float32)
    asrc = jnp.dot(xw, ae_ref[...], preferred_element_type=jnp.float32)
    adst = jnp.dot(xw, be_ref[...], preferred_element_type=jnp.float32)
    table_ref[...] = jnp.concatenate([xw, asrc], axis=1)
    ad_ref[...] = adst
    w16 = jnp.exp(_leaky(asrc + adst))
    w64 = jnp.dot(w16, xp_ref[...], preferred_element_type=jnp.float32)
    self_ref[...] = jnp.concatenate([xw * w64, w16], axis=1)


def _tc_mid_body(acc_ref, self_ref, b1_ref, w2p_ref, sv32_ref, dv16_ref,
                 xp_ref, table_ref, ad_ref, self2_ref):
    a = acc_ref[0] + acc_ref[1] + self_ref[...]
    den = jnp.dot(a[:, 64:], xp_ref[...], preferred_element_type=jnp.float32)
    h = a[:, :64] / (den + 1e-16) + b1_ref[...]
    h = jnp.where(h > 0.0, h, jnp.exp(jnp.minimum(h, 0.0)) - 1.0)
    xw2 = jnp.dot(h, w2p_ref[...], preferred_element_type=jnp.float32)
    t2 = xw2 + jnp.dot(xw2, sv32_ref[...], preferred_element_type=jnp.float32)
    adx = jnp.dot(xw2, dv16_ref[...], preferred_element_type=jnp.float32)
    table_ref[...] = t2
    ad_ref[...] = adx
    w = jnp.exp(_leaky(t2[:, 16:] + adx))
    self2_ref[...] = jnp.concatenate([xw2[:, :16] * w, w], axis=1)


def _tc_post_body(acc_ref, self_ref, b2_ref, o_ref):
    a = acc_ref[0] + acc_ref[1] + self_ref[...]
    o = a[:, 0:2] / (a[:, 16:18] + 1e-16) + b2_ref[...]
    m = jnp.max(o, axis=1, keepdims=True)
    o_ref[...] = o - m - jnp.log(jnp.sum(jnp.exp(o - m), axis=1, keepdims=True))


def _compact_att(a, pad_to=16):
    hh, cc = a.shape
    eye = jnp.eye(hh, dtype=a.dtype)
    t = eye[:, None, :] * a.T[None, :, :]
    m = t.reshape(hh * cc, hh)
    return jnp.pad(m, ((0, 0), (0, pad_to - hh)))


def kernel(x, edge_index, W1, att_src1, att_dst1, bias1,
           W2, att_src2, att_dst2, bias2):
    n, f_in = x.shape
    e = edge_index.shape[1]
    h1, c1 = att_src1.shape[1], att_src1.shape[2]
    c2 = att_src2.shape[2]
    d1 = h1 * c1

    ae1 = _compact_att(att_src1.reshape(h1, c1))
    be1 = _compact_att(att_dst1.reshape(h1, c1))
    xp = (jnp.arange(d1)[None, :] // c1
          == jnp.arange(16)[:, None]).astype(jnp.float32)
    w2p = jnp.zeros((d1, 32), jnp.float32).at[:, :c2].set(W2)
    sv = jnp.zeros((32,), jnp.float32).at[:c2].set(att_src2.reshape(c2))
    dv = jnp.zeros((32,), jnp.float32).at[:c2].set(att_dst2.reshape(c2))
    sv32 = jnp.concatenate(
        [jnp.zeros((32, 16), jnp.float32),
         jnp.broadcast_to(sv[:, None], (32, 16))], axis=1)
    dv16 = jnp.broadcast_to(dv[:, None], (32, 16))
    b1r = bias1.reshape(1, d1)
    b2r = bias2.reshape(1, c2)

    epw = e // _NW
    ch = _pick_chunk(epw)
    nch = epw // ch
    src3 = edge_index[0].reshape(_NW, nch, ch)
    dst3 = edge_index[1].reshape(_NW, nch, ch)

    r = 1000
    grid = (n // r,)
    wt1 = d1 + 16
    table1, ad1, self1 = pl.pallas_call(
        _tc_pre_body,
        grid=grid,
        in_specs=[
            pl.BlockSpec((r, f_in), lambda i: (i, 0)),
            pl.BlockSpec((f_in, d1), lambda i: (0, 0)),
            pl.BlockSpec((d1, 16), lambda i: (0, 0)),
            pl.BlockSpec((d1, 16), lambda i: (0, 0)),
            pl.BlockSpec((16, d1), lambda i: (0, 0)),
        ],
        out_specs=[
            pl.BlockSpec((r, wt1), lambda i: (i, 0)),
            pl.BlockSpec((r, 16), lambda i: (i, 0)),
            pl.BlockSpec((r, wt1), lambda i: (i, 0)),
        ],
        out_shape=[
            jax.ShapeDtypeStruct((n, wt1), jnp.float32),
            jax.ShapeDtypeStruct((n, 16), jnp.float32),
            jax.ShapeDtypeStruct((n, wt1), jnp.float32),
        ],
    )(x, W1, ae1, be1, xp)

    acc1 = _make_sc_edge(n, e, d1)(table1, ad1, src3, dst3)

    table2, ad2, self2 = pl.pallas_call(
        _tc_mid_body,
        grid=grid,
        in_specs=[
            pl.BlockSpec((_NC, r, wt1), lambda i: (0, i, 0)),
            pl.BlockSpec((r, wt1), lambda i: (i, 0)),
            pl.BlockSpec((1, d1), lambda i: (0, 0)),
            pl.BlockSpec((d1, 32), lambda i: (0, 0)),
            pl.BlockSpec((32, 32), lambda i: (0, 0)),
            pl.BlockSpec((32, 16), lambda i: (0, 0)),
            pl.BlockSpec((16, d1), lambda i: (0, 0)),
        ],
        out_specs=[
            pl.BlockSpec((r, 32), lambda i: (i, 0)),
            pl.BlockSpec((r, 16), lambda i: (i, 0)),
            pl.BlockSpec((r, 32), lambda i: (i, 0)),
        ],
        out_shape=[
            jax.ShapeDtypeStruct((n, 32), jnp.float32),
            jax.ShapeDtypeStruct((n, 16), jnp.float32),
            jax.ShapeDtypeStruct((n, 32), jnp.float32),
        ],
    )(acc1, self1, b1r, w2p, sv32, dv16, xp)

    acc2 = _make_sc_edge(n, e, 16)(table2, ad2, src3, dst3)

    out = pl.pallas_call(
        _tc_post_body,
        grid=grid,
        in_specs=[
            pl.BlockSpec((_NC, r, 32), lambda i: (0, i, 0)),
            pl.BlockSpec((r, 32), lambda i: (i, 0)),
            pl.BlockSpec((1, c2), lambda i: (0, 0)),
        ],
        out_specs=pl.BlockSpec((r, c2), lambda i: (i, 0)),
        out_shape=jax.ShapeDtypeStruct((n, c2), jnp.float32),
    )(acc2, self2, b2r)

    return out
